# Initial kernel scaffold; baseline (speedup 1.0000x reference)
import jax
import jax.numpy as jnp
from jax.experimental import pallas as pl


def kernel(input, tok_table, pos_table):
    raise NotImplementedError("write your pallas kernel here")


# speedup vs baseline: 3.9693x; 3.9693x over previous
import functools

import jax
import jax.numpy as jnp
from jax.experimental import pallas as pl
from jax.experimental.pallas import tpu as pltpu
from jax.experimental.pallas import tpu_sc as plsc

LANES = 16
WIN = 400
SUB = 80


def kernel(input, tok_table, pos_table):
    batch, seqlen = input.shape
    vocab, hdim = tok_table.shape
    n = batch * seqlen
    idx = input.reshape(n).astype(jnp.int32)

    mesh = plsc.VectorSubcoreMesh(core_axis_name="core",
                                  subcore_axis_name="subcore")

    @functools.partial(
        pl.kernel,
        out_type=jax.ShapeDtypeStruct((n, hdim), jnp.float32),
        mesh=mesh,
        compiler_params=pltpu.CompilerParams(use_tc_tiling_on_sc=False),
        scratch_types=[
            pltpu.VMEM((pos_table.shape[0], hdim), jnp.float32),
            pltpu.SemaphoreType.DMA,
        ],
    )
    def emb(tok_hbm, idx_hbm, pos_hbm, out_hbm, pos_vmem, sem):
        pltpu.sync_copy(pos_hbm, pos_vmem)

        def body(i_vmem, o_vmem):
            copies = [
                pltpu.async_copy(
                    tok_hbm.at[i_vmem.at[pl.ds(s * SUB, SUB)]],
                    o_vmem.at[pl.ds(s * SUB, SUB)],
                    sem,
                )
                for s in range(WIN // SUB)
            ]
            for c in copies:
                c.wait()

            @pl.loop(0, seqlen)
            def _(l):
                pos_vecs = [pos_vmem[l + 1, pl.ds(k * LANES, LANES)]
                            for k in range(hdim // LANES)]
                for j in range(WIN // seqlen):
                    for k in range(hdim // LANES):
                        o_vmem[j * seqlen + l, pl.ds(k * LANES, LANES)] += pos_vecs[k]

        pltpu.emit_pipeline(
            body,
            grid=(n // WIN,),
            in_specs=[pl.BlockSpec((WIN,), lambda i: (i,))],
            out_specs=[pl.BlockSpec((WIN, hdim), lambda i: (i, 0))],
            core_axis_name=("core", "subcore"),
            dimension_semantics=(pltpu.PARALLEL,),
        )(idx_hbm, out_hbm)

    out = emb(tok_table, idx, pos_table)
    return out.reshape(batch, seqlen, hdim)


# speedup vs baseline: 5.7199x; 1.4411x over previous
import functools

import jax
import jax.numpy as jnp
from jax.experimental import pallas as pl
from jax.experimental.pallas import tpu as pltpu
from jax.experimental.pallas import tpu_sc as plsc

LANES = 16
WIN = 800
SUB = 80


def kernel(input, tok_table, pos_table):
    batch, seqlen = input.shape
    vocab, hdim = tok_table.shape
    n = batch * seqlen
    idx = input.reshape(n).astype(jnp.int32)

    mesh = plsc.VectorSubcoreMesh(core_axis_name="core",
                                  subcore_axis_name="subcore")

    @functools.partial(
        pl.kernel,
        out_type=jax.ShapeDtypeStruct((n, hdim), jnp.float32),
        mesh=mesh,
        compiler_params=pltpu.CompilerParams(use_tc_tiling_on_sc=False),
        scratch_types=[
            pltpu.VMEM((pos_table.shape[0], hdim), jnp.float32),
            pltpu.SemaphoreType.DMA,
        ],
    )
    def emb(tok_hbm, idx_hbm, pos_hbm, out_hbm, pos_vmem, sem):
        pltpu.sync_copy(pos_hbm, pos_vmem)

        def body(i_vmem, o_vmem):
            copies = [
                pltpu.async_copy(
                    tok_hbm.at[i_vmem.at[pl.ds(s * SUB, SUB)]],
                    o_vmem.at[pl.ds(s * SUB, SUB)],
                    sem,
                )
                for s in range(WIN // SUB)
            ]
            for c in copies:
                c.wait()

            @pl.loop(0, seqlen)
            def _(l):
                pos_vecs = [pos_vmem[l + 1, pl.ds(k * LANES, LANES)]
                            for k in range(hdim // LANES)]
                for j in range(WIN // seqlen):
                    for k in range(hdim // LANES):
                        plsc.addupdate(
                            o_vmem.at[j * seqlen + l, pl.ds(k * LANES, LANES)],
                            pos_vecs[k])

        pltpu.emit_pipeline(
            body,
            grid=(n // WIN,),
            in_specs=[pl.BlockSpec((WIN,), lambda i: (i,))],
            out_specs=[pl.BlockSpec((WIN, hdim), lambda i: (i, 0))],
            core_axis_name=("core", "subcore"),
            dimension_semantics=(pltpu.PARALLEL,),
        )(idx_hbm, out_hbm)

    out = emb(tok_table, idx, pos_table)
    return out.reshape(batch, seqlen, hdim)
